# baseline (device time: 67027 ns/iter reference)
import numpy as np
import jax
import jax.numpy as jnp
from jax import lax
from jax.experimental import pallas as pl
from jax.experimental.pallas import tpu as pltpu

N_DEV = 8
B = 2
SQ = 256
D = 768
HC = 4
DH = 64
CW = HC * DH
BSQ = B * SQ

_sem_signal = getattr(pl, "semaphore_signal", None) or pltpu.semaphore_signal
_sem_wait = getattr(pl, "semaphore_wait", None) or pltpu.semaphore_wait
_CompilerParams = getattr(pltpu, "CompilerParams", None) or getattr(
    pltpu, "TPUCompilerParams"
)


def _consts():
    inv = 1.0 / (10000.0 ** (np.arange(0, DH, 2) / DH))
    pos = np.arange(SQ)[:, None] * inv[None, :]
    cos = np.repeat(np.cos(pos), 2, axis=-1)
    sin = np.repeat(np.sin(pos), 2, axis=-1)
    a = np.sqrt(0.125)
    cosm = np.tile(cos * a, (B, HC)).astype(np.float32)
    sinm = np.tile(sin * a, (B, HC)).astype(np.float32)
    r = np.zeros((DH, DH), np.float32)
    for i in range(0, DH, 2):
        r[i + 1, i] = -1.0
        r[i, i + 1] = 1.0
    rot = np.kron(np.eye(HC, dtype=np.float32), r)
    return cosm, sinm, rot


_COS, _SIN, _ROT = _consts()

import os as _os
_COMPUTE_ONLY = _os.environ.get("SCBAND_COMPUTE_ONLY") == "1"


def kernel(x, Wq, Wk, Wv, Wo):
    bf16 = jnp.bfloat16
    f32 = jnp.float32

    def body(x_ref, wq_ref, wk_ref, wv_ref, wo_ref, cos_ref, sin_ref,
             rot_ref, out_ref, xb, wbuf, obuf,
             s_r, r_l, s_l, r_r, s_z, r_z):
        me = lax.axis_index("i")
        base = (me // 4) * 4
        pp = me - base
        right = base + lax.rem(pp + 1, 4)
        left = base + lax.rem(pp + 3, 4)
        partner = lax.rem(me + 4, N_DEV)

        barrier = pltpu.get_barrier_semaphore()
        for nbr in (left, right, partner):
            _sem_signal(barrier, inc=1, device_id=(nbr,),
                        device_id_type=pl.DeviceIdType.MESH)
        _sem_wait(barrier, 3)

        xb[0:SQ, :] = x_ref[0].astype(bf16)
        xb[SQ:BSQ, :] = x_ref[1].astype(bf16)
        wbuf[0, 0] = wq_ref[...].astype(bf16)
        wbuf[0, 1] = wk_ref[...].astype(bf16)
        wbuf[0, 2] = wv_ref[...].astype(bf16)
        obuf[0] = wo_ref[...].astype(bf16)

        cosm = cos_ref[...]
        sinm = sin_ref[...]
        rotm = rot_ref[...]

        def compute(slot, first=False):
            xv = xb[...]
            w3 = wbuf[slot]
            wo = obuf[slot]
            q = jnp.dot(xv, w3[0], preferred_element_type=f32).astype(bf16)
            k = jnp.dot(xv, w3[1], preferred_element_type=f32).astype(bf16)
            vb = jnp.dot(xv, w3[2], preferred_element_type=f32).astype(bf16)
            qr = q * cosm + jnp.dot(
                q, rotm, preferred_element_type=f32).astype(bf16) * sinm
            kr = k * cosm + jnp.dot(
                k, rotm, preferred_element_type=f32).astype(bf16) * sinm
            ctxs = []
            for b in range(B):
                row = slice(b * SQ, (b + 1) * SQ)
                cols = []
                for hh in range(HC):
                    col = slice(hh * DH, (hh + 1) * DH)
                    s = lax.dot_general(
                        qr[row, col], kr[row, col],
                        (((1,), (1,)), ((), ())),
                        preferred_element_type=f32)
                    e = jnp.exp(s)
                    ctx_u = jnp.dot(e.astype(bf16), vb[row, col],
                                    preferred_element_type=f32)
                    r = 1.0 / jnp.sum(e, axis=-1, keepdims=True)
                    cols.append(ctx_u * r)
                ctxs.append(jnp.concatenate(cols, axis=1))
            ctx = jnp.concatenate(ctxs, axis=0).astype(bf16)
            contrib = jnp.dot(ctx, wo, preferred_element_type=f32)
            for b in range(B):
                rows = contrib[b * SQ:(b + 1) * SQ, :]
                if first:
                    out_ref[b] = rows
                else:
                    out_ref[b] = out_ref[b] + rows

        def mk(src_slot, dst_slot, c, dst_dev, s_sem, r_sem):
            if c < 3:
                src, dst = wbuf.at[src_slot, c], wbuf.at[dst_slot, c]
            else:
                src, dst = obuf.at[src_slot], obuf.at[dst_slot]
            return pltpu.make_async_remote_copy(
                src_ref=src, dst_ref=dst, send_sem=s_sem, recv_sem=r_sem,
                device_id=(dst_dev,), device_id_type=pl.DeviceIdType.MESH)

        sent = []

        def send(src_slot, dst_slot, c, dst_dev, s_sems, r_sems, r, k):
            d = mk(src_slot, dst_slot, c, dst_dev,
                   s_sems.at[r, k], r_sems.at[r, k])
            d.start()
            sent.append(d)

        def recv(dst_slot, c, src_dev, r_sems, r, k):
            d = mk(dst_slot, dst_slot, c, src_dev,
                   s_z.at[0, 0], r_sems.at[r, k])
            d.wait_recv()

        if _COMPUTE_ONLY:
            compute(0, first=True)
            for _ in range(7):
                compute(0)
            return

        for c in range(4):
            send(0, 4, c, partner, s_z, r_z, 0, c)
            send(0, 3, c, right, s_r, r_l, 0, c)
            send(0, 1, c, left, s_l, r_r, 0, c)
        compute(0, first=True)

        for c in range(4):
            recv(4, c, partner, r_z, 0, c)
        send(4, 7, 0, right, s_r, r_l, 1, 0)
        send(4, 7, 1, right, s_r, r_l, 1, 1)
        send(4, 5, 2, left, s_l, r_r, 1, 0)
        send(4, 5, 3, left, s_l, r_r, 1, 1)
        for c in range(4):
            recv(3, c, left, r_l, 0, c)
        for c in range(4):
            recv(1, c, right, r_r, 0, c)
        send(3, 7, 2, partner, s_z, r_z, 1, 0)
        send(3, 7, 3, partner, s_z, r_z, 1, 1)
        send(1, 5, 0, partner, s_z, r_z, 1, 2)
        send(1, 5, 1, partner, s_z, r_z, 1, 3)
        compute(4)

        recv(7, 0, left, r_l, 1, 0)
        recv(7, 1, left, r_l, 1, 1)
        send(3, 2, 0, right, s_r, r_l, 2, 0)
        send(3, 2, 1, right, s_r, r_l, 2, 1)
        send(7, 6, 0, right, s_r, r_l, 2, 2)
        send(7, 6, 1, right, s_r, r_l, 2, 3)
        recv(5, 2, right, r_r, 1, 0)
        recv(5, 3, right, r_r, 1, 1)
        send(1, 2, 2, left, s_l, r_r, 2, 0)
        send(1, 2, 3, left, s_l, r_r, 2, 1)
        send(5, 6, 2, left, s_l, r_r, 2, 2)
        send(5, 6, 3, left, s_l, r_r, 2, 3)
        compute(3)
        compute(1)

        recv(7, 2, partner, r_z, 1, 0)
        recv(7, 3, partner, r_z, 1, 1)
        recv(5, 0, partner, r_z, 1, 2)
        recv(5, 1, partner, r_z, 1, 3)
        compute(7)
        compute(5)

        for k, (slot, c) in enumerate([(2, 0), (2, 1), (6, 0), (6, 1)]):
            recv(slot, c, left, r_l, 2, k)
        for k, (slot, c) in enumerate([(2, 2), (2, 3), (6, 2), (6, 3)]):
            recv(slot, c, right, r_r, 2, k)
        compute(2)
        compute(6)

        for d in sent:
            d.wait_send()

    out_shape = jax.ShapeDtypeStruct((B, SQ, D), f32)
    cosm = jnp.asarray(_COS, dtype=bf16)
    sinm = jnp.asarray(_SIN, dtype=bf16)
    rotm = jnp.asarray(_ROT, dtype=bf16)
    return pl.pallas_call(
        body,
        out_shape=out_shape,
        in_specs=[pl.BlockSpec(memory_space=pltpu.VMEM)] * 8,
        out_specs=pl.BlockSpec(memory_space=pltpu.VMEM),
        scratch_shapes=[
            pltpu.VMEM((BSQ, D), bf16),
            pltpu.VMEM((N_DEV, 3, D, CW), bf16),
            pltpu.VMEM((N_DEV, CW, D), bf16),
            pltpu.SemaphoreType.DMA((3, 4)),
            pltpu.SemaphoreType.DMA((3, 4)),
            pltpu.SemaphoreType.DMA((3, 4)),
            pltpu.SemaphoreType.DMA((3, 4)),
            pltpu.SemaphoreType.DMA((2, 4)),
            pltpu.SemaphoreType.DMA((2, 4)),
        ],
        compiler_params=_CompilerParams(collective_id=0),
    )(x, Wq, Wk, Wv, Wo, cosm, sinm, rotm)


# device time: 64689 ns/iter; 1.0361x vs baseline; 1.0361x over previous
import numpy as np
import jax
import jax.numpy as jnp
from jax import lax
from jax.experimental import pallas as pl
from jax.experimental.pallas import tpu as pltpu

N_DEV = 8
B = 2
SQ = 256
D = 768
HC = 4
DH = 64
CW = HC * DH
BSQ = B * SQ

_sem_signal = getattr(pl, "semaphore_signal", None) or pltpu.semaphore_signal
_sem_wait = getattr(pl, "semaphore_wait", None) or pltpu.semaphore_wait
_CompilerParams = getattr(pltpu, "CompilerParams", None) or getattr(
    pltpu, "TPUCompilerParams"
)


def _consts():
    inv = 1.0 / (10000.0 ** (np.arange(0, DH, 2) / DH))
    pos = np.arange(SQ)[:, None] * inv[None, :]
    cos = np.repeat(np.cos(pos), 2, axis=-1)
    sin = np.repeat(np.sin(pos), 2, axis=-1)
    a = np.sqrt(0.125)
    cosm = np.tile(cos * a, (B, HC)).astype(np.float32)
    sinm = np.tile(sin * a, (B, HC)).astype(np.float32)
    r = np.zeros((DH, DH), np.float32)
    for i in range(0, DH, 2):
        r[i + 1, i] = -1.0
        r[i, i + 1] = 1.0
    rot = np.kron(np.eye(HC, dtype=np.float32), r)
    return cosm, sinm, rot


_COS, _SIN, _ROT = _consts()

import os as _os
_COMPUTE_ONLY = _os.environ.get("SCBAND_COMPUTE_ONLY") == "1"


def kernel(x, Wq, Wk, Wv, Wo):
    bf16 = jnp.bfloat16
    f32 = jnp.float32

    def body(x_ref, wq_ref, wk_ref, wv_ref, wo_ref, cos_ref, sin_ref,
             rot_ref, out_ref, xb, wbuf, obuf,
             s_r, r_l, s_l, r_r, s_z, r_z):
        me = lax.axis_index("i")
        base = (me // 4) * 4
        pp = me - base
        right = base + lax.rem(pp + 1, 4)
        left = base + lax.rem(pp + 3, 4)
        partner = lax.rem(me + 4, N_DEV)

        barrier = pltpu.get_barrier_semaphore()
        for nbr in (left, right, partner):
            _sem_signal(barrier, inc=1, device_id=(nbr,),
                        device_id_type=pl.DeviceIdType.MESH)
        _sem_wait(barrier, 3)

        xb[0:SQ, :] = x_ref[0].astype(bf16)
        xb[SQ:BSQ, :] = x_ref[1].astype(bf16)
        wbuf[0, 0] = wq_ref[...].astype(bf16)
        wbuf[0, 1] = wk_ref[...].astype(bf16)
        wbuf[0, 2] = wv_ref[...].astype(bf16)
        obuf[0] = wo_ref[...].astype(bf16)

        cosm = cos_ref[...]
        sinm = sin_ref[...]
        rotm = rot_ref[...]

        def compute(slot, first=False):
            xv = xb[...]
            w3 = wbuf[slot]
            wo = obuf[slot]
            q = jnp.dot(xv, w3[0], preferred_element_type=f32).astype(bf16)
            k = jnp.dot(xv, w3[1], preferred_element_type=f32).astype(bf16)
            vb = jnp.dot(xv, w3[2], preferred_element_type=f32).astype(bf16)
            qr = q * cosm + jnp.dot(
                q, rotm, preferred_element_type=f32).astype(bf16) * sinm
            kr = k * cosm + jnp.dot(
                k, rotm, preferred_element_type=f32).astype(bf16) * sinm
            ctxs = []
            for b in range(B):
                row = slice(b * SQ, (b + 1) * SQ)
                cols = []
                for hh in range(HC):
                    col = slice(hh * DH, (hh + 1) * DH)
                    s = lax.dot_general(
                        qr[row, col], kr[row, col],
                        (((1,), (1,)), ((), ())),
                        preferred_element_type=f32)
                    e = jnp.exp(s)
                    ctx_u = jnp.dot(e.astype(bf16), vb[row, col],
                                    preferred_element_type=f32)
                    r = 1.0 / jnp.sum(e, axis=-1, keepdims=True)
                    cols.append(ctx_u * r)
                ctxs.append(jnp.concatenate(cols, axis=1))
            ctx = jnp.concatenate(ctxs, axis=0).astype(bf16)
            contrib = jnp.dot(ctx, wo, preferred_element_type=f32)
            for b in range(B):
                rows = contrib[b * SQ:(b + 1) * SQ, :]
                if first:
                    out_ref[b] = rows
                else:
                    out_ref[b] = out_ref[b] + rows

        def mk(src_slot, dst_slot, c, dst_dev, s_sem, r_sem):
            if c < 3:
                src, dst = wbuf.at[src_slot, c], wbuf.at[dst_slot, c]
            else:
                src, dst = obuf.at[src_slot], obuf.at[dst_slot]
            return pltpu.make_async_remote_copy(
                src_ref=src, dst_ref=dst, send_sem=s_sem, recv_sem=r_sem,
                device_id=(dst_dev,), device_id_type=pl.DeviceIdType.MESH)

        sent = []

        def send(src_slot, dst_slot, c, dst_dev, s_sems, r_sems, r, k):
            d = mk(src_slot, dst_slot, c, dst_dev,
                   s_sems.at[r, k], r_sems.at[r, k])
            d.start()
            sent.append(d)

        def recv(dst_slot, c, src_dev, r_sems, r, k):
            d = mk(dst_slot, dst_slot, c, src_dev,
                   s_z.at[0, 0], r_sems.at[r, k])
            d.wait_recv()

        if _COMPUTE_ONLY:
            compute(0, first=True)
            for _ in range(7):
                compute(0)
            return

        for c in range(4):
            send(0, 4, c, partner, s_z, r_z, 0, c)
            send(0, 3, c, right, s_r, r_l, 0, c)
            send(0, 1, c, left, s_l, r_r, 0, c)
        compute(0, first=True)

        for c in range(4):
            recv(4, c, partner, r_z, 0, c)
        send(4, 7, 0, right, s_r, r_l, 1, 0)
        send(4, 7, 1, right, s_r, r_l, 1, 1)
        send(4, 5, 2, left, s_l, r_r, 1, 0)
        send(4, 5, 3, left, s_l, r_r, 1, 1)
        for c in range(4):
            recv(3, c, left, r_l, 0, c)
        for c in range(4):
            recv(1, c, right, r_r, 0, c)
        send(3, 7, 2, partner, s_z, r_z, 1, 0)
        send(3, 7, 3, partner, s_z, r_z, 1, 1)
        send(1, 5, 0, partner, s_z, r_z, 1, 2)
        send(1, 5, 1, partner, s_z, r_z, 1, 3)
        send(3, 2, 0, right, s_r, r_l, 2, 0)
        send(3, 2, 1, right, s_r, r_l, 2, 1)
        send(1, 2, 2, left, s_l, r_r, 2, 0)
        send(1, 2, 3, left, s_l, r_r, 2, 1)
        compute(4)
        compute(3)
        compute(1)

        recv(7, 0, left, r_l, 1, 0)
        recv(7, 1, left, r_l, 1, 1)
        recv(7, 2, partner, r_z, 1, 0)
        recv(7, 3, partner, r_z, 1, 1)
        send(7, 6, 0, right, s_r, r_l, 2, 2)
        send(7, 6, 1, right, s_r, r_l, 2, 3)
        compute(7)
        recv(5, 2, right, r_r, 1, 0)
        recv(5, 3, right, r_r, 1, 1)
        recv(5, 0, partner, r_z, 1, 2)
        recv(5, 1, partner, r_z, 1, 3)
        send(5, 6, 2, left, s_l, r_r, 2, 2)
        send(5, 6, 3, left, s_l, r_r, 2, 3)
        compute(5)

        for k, (slot, c) in enumerate([(2, 0), (2, 1)]):
            recv(slot, c, left, r_l, 2, k)
        for k, (slot, c) in enumerate([(2, 2), (2, 3)]):
            recv(slot, c, right, r_r, 2, k)
        compute(2)
        for k, (slot, c) in enumerate([(6, 0), (6, 1)], start=2):
            recv(slot, c, left, r_l, 2, k)
        for k, (slot, c) in enumerate([(6, 2), (6, 3)], start=2):
            recv(slot, c, right, r_r, 2, k)
        compute(6)

        for d in sent:
            d.wait_send()

    out_shape = jax.ShapeDtypeStruct((B, SQ, D), f32)
    cosm = jnp.asarray(_COS, dtype=bf16)
    sinm = jnp.asarray(_SIN, dtype=bf16)
    rotm = jnp.asarray(_ROT, dtype=bf16)
    return pl.pallas_call(
        body,
        out_shape=out_shape,
        in_specs=[pl.BlockSpec(memory_space=pltpu.VMEM)] * 8,
        out_specs=pl.BlockSpec(memory_space=pltpu.VMEM),
        scratch_shapes=[
            pltpu.VMEM((BSQ, D), bf16),
            pltpu.VMEM((N_DEV, 3, D, CW), bf16),
            pltpu.VMEM((N_DEV, CW, D), bf16),
            pltpu.SemaphoreType.DMA((3, 4)),
            pltpu.SemaphoreType.DMA((3, 4)),
            pltpu.SemaphoreType.DMA((3, 4)),
            pltpu.SemaphoreType.DMA((3, 4)),
            pltpu.SemaphoreType.DMA((2, 4)),
            pltpu.SemaphoreType.DMA((2, 4)),
        ],
        compiler_params=_CompilerParams(collective_id=0),
    )(x, Wq, Wk, Wv, Wo, cosm, sinm, rotm)


# device time: 46251 ns/iter; 1.4492x vs baseline; 1.3987x over previous
import os as _os

import numpy as np
import jax
import jax.numpy as jnp
from jax import lax
from jax.experimental import pallas as pl
from jax.experimental.pallas import tpu as pltpu

N_DEV = 8
B = 2
SQ = 256
D = 768
HC = 4
DH = 64
CW = HC * DH
BSQ = B * SQ

_sem_signal = getattr(pl, "semaphore_signal", None) or pltpu.semaphore_signal
_sem_wait = getattr(pl, "semaphore_wait", None) or pltpu.semaphore_wait
_CompilerParams = getattr(pltpu, "CompilerParams", None) or getattr(
    pltpu, "TPUCompilerParams"
)

_COMPUTE_ONLY = _os.environ.get("SCBAND_COMPUTE_ONLY") == "1"


def _consts():
    inv = 1.0 / (10000.0 ** (np.arange(0, DH, 2) / DH))
    pos = np.arange(SQ)[:, None] * inv[None, :]
    cos = np.repeat(np.cos(pos), 2, axis=-1)
    sin = np.repeat(np.sin(pos), 2, axis=-1)
    a = np.sqrt(0.125)
    cosm = np.tile(cos * a, (B, HC)).astype(np.float32)
    sinm = np.tile(sin * a, (B, HC)).astype(np.float32)
    r = np.zeros((DH, DH), np.float32)
    for i in range(0, DH, 2):
        r[i + 1, i] = -1.0
        r[i, i + 1] = 1.0
    rot = np.kron(np.eye(HC, dtype=np.float32), r)
    return cosm, sinm, rot


_COS, _SIN, _ROT = _consts()


def kernel(x, Wq, Wk, Wv, Wo):
    bf16 = jnp.bfloat16
    f32 = jnp.float32
    i8 = jnp.int8

    def body(x_ref, wq_ref, wk_ref, wv_ref, wo_ref, cos_ref, sin_ref,
             rot_ref, out_ref, xb, wbuf, obuf, scb,
             s_r, r_l, s_l, r_r, s_z, r_z,
             t_r, u_l, t_l, u_r, t_z, u_z):
        me = lax.axis_index("i")
        base = (me // 4) * 4
        pp = me - base
        right = base + lax.rem(pp + 1, 4)
        left = base + lax.rem(pp + 3, 4)
        partner = lax.rem(me + 4, N_DEV)

        barrier = pltpu.get_barrier_semaphore()
        for nbr in (left, right, partner):
            _sem_signal(barrier, inc=1, device_id=(nbr,),
                        device_id_type=pl.DeviceIdType.MESH)
        _sem_wait(barrier, 3)

        cosm = cos_ref[...]
        sinm = sin_ref[...]
        rotm = rot_ref[...]

        xb[0:SQ, :] = x_ref[0].astype(bf16)
        xb[SQ:BSQ, :] = x_ref[1].astype(bf16)

        def quant(w, pair):
            a = jnp.max(jnp.abs(w), axis=0, keepdims=True)
            if pair:
                ab = a.astype(bf16)
                swap = jnp.dot(ab, jnp.abs(rotm),
                               preferred_element_type=f32).astype(bf16)
                a = jnp.maximum(ab, swap).astype(f32)
            a = jnp.maximum(a, 1e-20)
            qi = jnp.clip(jnp.round(w * (127.0 / a)),
                          -127.0, 127.0).astype(i8)
            return qi, a * (1.0 / 127.0)

        qq, sq0 = quant(wq_ref[...], True)
        qk, sk0 = quant(wk_ref[...], True)
        qv, sv0 = quant(wv_ref[...], False)
        qo, so0 = quant(wo_ref[...], False)
        wbuf[0, 0] = qq
        wbuf[0, 1] = qk
        wbuf[0, 2] = qv
        obuf[0] = qo
        scb[0, 0, 0:CW] = sq0[0]
        scb[0, 1, 0:CW] = sk0[0]
        scb[0, 2, 0:CW] = sv0[0]
        scb[0, 3, :] = so0[0]

        def compute(slot, first=False):
            xv = xb[...]
            w3 = wbuf[slot]
            wo_i = obuf[slot]
            sq = scb[slot, 0, 0:CW]
            sk = scb[slot, 1, 0:CW]
            sv = scb[slot, 2, 0:CW]
            so = scb[slot, 3, :]
            q = jnp.dot(xv, w3[0].astype(bf16),
                        preferred_element_type=f32).astype(bf16)
            k = jnp.dot(xv, w3[1].astype(bf16),
                        preferred_element_type=f32).astype(bf16)
            vb = jnp.dot(xv, w3[2].astype(bf16),
                         preferred_element_type=f32).astype(bf16)
            vb = vb * sv.astype(bf16)[None, :]
            qr = q * cosm + jnp.dot(
                q, rotm, preferred_element_type=f32).astype(bf16) * sinm
            kr = k * cosm + jnp.dot(
                k, rotm, preferred_element_type=f32).astype(bf16) * sinm
            qr = qr * (sq * sk).astype(bf16)[None, :]
            ctxs = []
            for b in range(B):
                row = slice(b * SQ, (b + 1) * SQ)
                cols = []
                for hh in range(HC):
                    col = slice(hh * DH, (hh + 1) * DH)
                    s = lax.dot_general(
                        qr[row, col], kr[row, col],
                        (((1,), (1,)), ((), ())),
                        preferred_element_type=f32)
                    e = jnp.exp(s)
                    ctx_u = jnp.dot(e.astype(bf16), vb[row, col],
                                    preferred_element_type=f32)
                    r = 1.0 / jnp.sum(e, axis=-1, keepdims=True)
                    cols.append(ctx_u * r)
                ctxs.append(jnp.concatenate(cols, axis=1))
            ctx = jnp.concatenate(ctxs, axis=0).astype(bf16)
            wo_bf = wo_i.astype(bf16) * so.astype(bf16)[None, :]
            contrib = jnp.dot(ctx, wo_bf, preferred_element_type=f32)
            for b in range(B):
                rows = contrib[b * SQ:(b + 1) * SQ, :]
                if first:
                    out_ref[b] = rows
                else:
                    out_ref[b] = out_ref[b] + rows

        if _COMPUTE_ONLY:
            compute(0, first=True)
            for _ in range(7):
                compute(0)
            return

        def mk(src_slot, dst_slot, c, dst_dev, s_sem, r_sem):
            if c < 3:
                src, dst = wbuf.at[src_slot, c], wbuf.at[dst_slot, c]
            else:
                src, dst = obuf.at[src_slot], obuf.at[dst_slot]
            return pltpu.make_async_remote_copy(
                src_ref=src, dst_ref=dst, send_sem=s_sem, recv_sem=r_sem,
                device_id=(dst_dev,), device_id_type=pl.DeviceIdType.MESH)

        def mk_sc(src_slot, dst_slot, c, dst_dev, s_sem, r_sem):
            return pltpu.make_async_remote_copy(
                src_ref=scb.at[src_slot, c], dst_ref=scb.at[dst_slot, c],
                send_sem=s_sem, recv_sem=r_sem,
                device_id=(dst_dev,), device_id_type=pl.DeviceIdType.MESH)

        sent = []

        def send(src_slot, dst_slot, c, dst_dev, spair, rpair, r, k):
            dsc = mk_sc(src_slot, dst_slot, c, dst_dev,
                        spair[1].at[r, k], rpair[1].at[r, k])
            dsc.start()
            d = mk(src_slot, dst_slot, c, dst_dev,
                   spair[0].at[r, k], rpair[0].at[r, k])
            d.start()
            sent.append(dsc)
            sent.append(d)

        def recv(dst_slot, c, src_dev, rpair, r, k):
            mk_sc(dst_slot, dst_slot, c, src_dev,
                  t_z.at[0, 0], rpair[1].at[r, k]).wait_recv()
            mk(dst_slot, dst_slot, c, src_dev,
               s_z.at[0, 0], rpair[0].at[r, k]).wait_recv()

        S_R, S_L, S_Z = (s_r, t_r), (s_l, t_l), (s_z, t_z)
        R_L, R_R, R_Z = (r_l, u_l), (r_r, u_r), (r_z, u_z)

        for c in range(4):
            send(0, 4, c, partner, S_Z, R_Z, 0, c)
            send(0, 3, c, right, S_R, R_L, 0, c)
            send(0, 1, c, left, S_L, R_R, 0, c)
        compute(0, first=True)

        for c in range(4):
            recv(4, c, partner, R_Z, 0, c)
        send(4, 7, 0, right, S_R, R_L, 1, 0)
        send(4, 7, 1, right, S_R, R_L, 1, 1)
        send(4, 5, 2, left, S_L, R_R, 1, 0)
        send(4, 5, 3, left, S_L, R_R, 1, 1)
        for c in range(4):
            recv(3, c, left, R_L, 0, c)
        for c in range(4):
            recv(1, c, right, R_R, 0, c)
        send(3, 7, 2, partner, S_Z, R_Z, 1, 0)
        send(3, 7, 3, partner, S_Z, R_Z, 1, 1)
        send(1, 5, 0, partner, S_Z, R_Z, 1, 2)
        send(1, 5, 1, partner, S_Z, R_Z, 1, 3)
        send(3, 2, 0, right, S_R, R_L, 2, 0)
        send(3, 2, 1, right, S_R, R_L, 2, 1)
        send(1, 2, 2, left, S_L, R_R, 2, 0)
        send(1, 2, 3, left, S_L, R_R, 2, 1)
        compute(4)
        compute(3)
        compute(1)

        recv(7, 0, left, R_L, 1, 0)
        recv(7, 1, left, R_L, 1, 1)
        recv(7, 2, partner, R_Z, 1, 0)
        recv(7, 3, partner, R_Z, 1, 1)
        send(7, 6, 0, right, S_R, R_L, 2, 2)
        send(7, 6, 1, right, S_R, R_L, 2, 3)
        compute(7)
        recv(5, 2, right, R_R, 1, 0)
        recv(5, 3, right, R_R, 1, 1)
        recv(5, 0, partner, R_Z, 1, 2)
        recv(5, 1, partner, R_Z, 1, 3)
        send(5, 6, 2, left, S_L, R_R, 2, 2)
        send(5, 6, 3, left, S_L, R_R, 2, 3)
        compute(5)

        for k, (slot, c) in enumerate([(2, 0), (2, 1)]):
            recv(slot, c, left, R_L, 2, k)
        for k, (slot, c) in enumerate([(2, 2), (2, 3)]):
            recv(slot, c, right, R_R, 2, k)
        compute(2)
        for k, (slot, c) in enumerate([(6, 0), (6, 1)], start=2):
            recv(slot, c, left, R_L, 2, k)
        for k, (slot, c) in enumerate([(6, 2), (6, 3)], start=2):
            recv(slot, c, right, R_R, 2, k)
        compute(6)

        for d in sent:
            d.wait_send()

    out_shape = jax.ShapeDtypeStruct((B, SQ, D), f32)
    cosm = jnp.asarray(_COS, dtype=bf16)
    sinm = jnp.asarray(_SIN, dtype=bf16)
    rotm = jnp.asarray(_ROT, dtype=bf16)
    return pl.pallas_call(
        body,
        out_shape=out_shape,
        in_specs=[pl.BlockSpec(memory_space=pltpu.VMEM)] * 8,
        out_specs=pl.BlockSpec(memory_space=pltpu.VMEM),
        scratch_shapes=[
            pltpu.VMEM((BSQ, D), bf16),
            pltpu.VMEM((N_DEV, 3, D, CW), i8),
            pltpu.VMEM((N_DEV, CW, D), i8),
            pltpu.VMEM((N_DEV, 4, D), f32),
            pltpu.SemaphoreType.DMA((3, 4)),
            pltpu.SemaphoreType.DMA((3, 4)),
            pltpu.SemaphoreType.DMA((3, 4)),
            pltpu.SemaphoreType.DMA((3, 4)),
            pltpu.SemaphoreType.DMA((2, 4)),
            pltpu.SemaphoreType.DMA((2, 4)),
            pltpu.SemaphoreType.DMA((3, 4)),
            pltpu.SemaphoreType.DMA((3, 4)),
            pltpu.SemaphoreType.DMA((3, 4)),
            pltpu.SemaphoreType.DMA((3, 4)),
            pltpu.SemaphoreType.DMA((2, 4)),
            pltpu.SemaphoreType.DMA((2, 4)),
        ],
        compiler_params=_CompilerParams(collective_id=0),
    )(x, Wq, Wk, Wv, Wo, cosm, sinm, rotm)


# device time: 44392 ns/iter; 1.5099x vs baseline; 1.0419x over previous
import os as _os

import numpy as np
import jax
import jax.numpy as jnp
from jax import lax
from jax.experimental import pallas as pl
from jax.experimental.pallas import tpu as pltpu

N_DEV = 8
B = 2
SQ = 256
D = 768
HC = 4
DH = 64
CW = HC * DH
BSQ = B * SQ

_sem_signal = getattr(pl, "semaphore_signal", None) or pltpu.semaphore_signal
_sem_wait = getattr(pl, "semaphore_wait", None) or pltpu.semaphore_wait
_CompilerParams = getattr(pltpu, "CompilerParams", None) or getattr(
    pltpu, "TPUCompilerParams"
)

_COMPUTE_ONLY = _os.environ.get("SCBAND_COMPUTE_ONLY") == "1"


def _consts():
    inv = 1.0 / (10000.0 ** (np.arange(0, DH, 2) / DH))
    pos = np.arange(SQ)[:, None] * inv[None, :]
    cos = np.repeat(np.cos(pos), 2, axis=-1)
    sin = np.repeat(np.sin(pos), 2, axis=-1)
    a = np.sqrt(0.125)
    cosm = np.tile(cos * a, (B, HC)).astype(np.float32)
    sinm = np.tile(sin * a, (B, HC)).astype(np.float32)
    r = np.zeros((DH, DH), np.float32)
    for i in range(0, DH, 2):
        r[i + 1, i] = -1.0
        r[i, i + 1] = 1.0
    rot = np.kron(np.eye(HC, dtype=np.float32), r)
    return cosm, sinm, rot


_COS, _SIN, _ROT = _consts()


def kernel(x, Wq, Wk, Wv, Wo):
    bf16 = jnp.bfloat16
    f32 = jnp.float32
    i8 = jnp.int8

    def body(x_ref, wq_ref, wk_ref, wv_ref, wo_ref, cos_ref, sin_ref,
             rot_ref, out_ref, xb, wbuf, obuf, scb,
             s_r, r_l, s_l, r_r, s_z, r_z,
             t_r, u_l, t_l, u_r, t_z, u_z):
        me = lax.axis_index("i")
        base = (me // 4) * 4
        pp = me - base
        right = base + lax.rem(pp + 1, 4)
        left = base + lax.rem(pp + 3, 4)
        partner = lax.rem(me + 4, N_DEV)

        barrier = pltpu.get_barrier_semaphore()
        for nbr in (left, right, partner):
            _sem_signal(barrier, inc=1, device_id=(nbr,),
                        device_id_type=pl.DeviceIdType.MESH)
        _sem_wait(barrier, 3)

        cosm = cos_ref[...]
        sinm = sin_ref[...]
        rotm = rot_ref[...]

        xb[0:SQ, :] = x_ref[0].astype(bf16)
        xb[SQ:BSQ, :] = x_ref[1].astype(bf16)

        def quant(w, pair):
            a = jnp.max(jnp.abs(w), axis=0, keepdims=True)
            if pair:
                ab = a.astype(bf16)
                swap = jnp.dot(ab, jnp.abs(rotm),
                               preferred_element_type=f32).astype(bf16)
                a = jnp.maximum(ab, swap).astype(f32)
            a = jnp.maximum(a, 1e-20)
            qi = jnp.clip(jnp.round(w * (127.0 / a)),
                          -127.0, 127.0).astype(i8)
            return qi, a * (1.0 / 127.0)

        def stage_comp(c):
            if c < 3:
                ref, pair = ((wq_ref, True), (wk_ref, True),
                             (wv_ref, False))[c]
                qi, sc = quant(ref[...], pair)
                wbuf[0, c] = qi
                scb[0, c, :] = sc[0]
            else:
                w = wo_ref[...]
                a = jnp.maximum(jnp.max(jnp.abs(w), axis=1, keepdims=True),
                                1e-20)
                obuf[0] = jnp.clip(jnp.round(w * (127.0 / a)),
                                   -127.0, 127.0).astype(i8)
                scb[0, 3, :] = a[:, 0] * (1.0 / 127.0)

        def compute(slot, first=False):
            xv = xb[...]
            w3 = wbuf[slot]
            wo_i = obuf[slot]
            sq = scb[slot, 0, :]
            sk = scb[slot, 1, :]
            sv = scb[slot, 2, :]
            so = scb[slot, 3, :]
            q = jnp.dot(xv, w3[0].astype(bf16),
                        preferred_element_type=f32).astype(bf16)
            k = jnp.dot(xv, w3[1].astype(bf16),
                        preferred_element_type=f32).astype(bf16)
            vb = jnp.dot(xv, w3[2].astype(bf16),
                         preferred_element_type=f32).astype(bf16)
            vb = vb * sv.astype(bf16)[None, :]
            qr = q * cosm + jnp.dot(
                q, rotm, preferred_element_type=f32).astype(bf16) * sinm
            kr = k * cosm + jnp.dot(
                k, rotm, preferred_element_type=f32).astype(bf16) * sinm
            qr = qr * (sq * sk).astype(bf16)[None, :]
            ctxs = []
            for b in range(B):
                row = slice(b * SQ, (b + 1) * SQ)
                cols = []
                for hh in range(HC):
                    col = slice(hh * DH, (hh + 1) * DH)
                    s = lax.dot_general(
                        qr[row, col], kr[row, col],
                        (((1,), (1,)), ((), ())),
                        preferred_element_type=f32)
                    e = jnp.exp(s)
                    ctx_u = jnp.dot(e.astype(bf16), vb[row, col],
                                    preferred_element_type=f32)
                    r = 1.0 / jnp.sum(e, axis=-1, keepdims=True)
                    cols.append(ctx_u * r)
                ctxs.append(jnp.concatenate(cols, axis=1))
            ctx = jnp.concatenate(ctxs, axis=0).astype(bf16)
            ctx = ctx * so.astype(bf16)[None, :]
            contrib = jnp.dot(ctx, wo_i.astype(bf16),
                              preferred_element_type=f32)
            for b in range(B):
                rows = contrib[b * SQ:(b + 1) * SQ, :]
                if first:
                    out_ref[b] = rows
                else:
                    out_ref[b] = out_ref[b] + rows

        if _COMPUTE_ONLY:
            for c in range(4):
                stage_comp(c)
            compute(0, first=True)
            for _ in range(7):
                compute(0)
            return

        def mk(src_slot, dst_slot, c, dst_dev, s_sem, r_sem):
            if c < 3:
                src, dst = wbuf.at[src_slot, c], wbuf.at[dst_slot, c]
            else:
                src, dst = obuf.at[src_slot], obuf.at[dst_slot]
            return pltpu.make_async_remote_copy(
                src_ref=src, dst_ref=dst, send_sem=s_sem, recv_sem=r_sem,
                device_id=(dst_dev,), device_id_type=pl.DeviceIdType.MESH)

        def mk_sc(src_slot, dst_slot, c, dst_dev, s_sem, r_sem):
            return pltpu.make_async_remote_copy(
                src_ref=scb.at[src_slot, c], dst_ref=scb.at[dst_slot, c],
                send_sem=s_sem, recv_sem=r_sem,
                device_id=(dst_dev,), device_id_type=pl.DeviceIdType.MESH)

        sent = []

        def send(src_slot, dst_slot, c, dst_dev, spair, rpair, r, k):
            dsc = mk_sc(src_slot, dst_slot, c, dst_dev,
                        spair[1].at[r, k], rpair[1].at[r, k])
            dsc.start()
            d = mk(src_slot, dst_slot, c, dst_dev,
                   spair[0].at[r, k], rpair[0].at[r, k])
            d.start()
            sent.append(dsc)
            sent.append(d)

        def recv(dst_slot, c, src_dev, rpair, r, k):
            mk_sc(dst_slot, dst_slot, c, src_dev,
                  t_z.at[0, 0], rpair[1].at[r, k]).wait_recv()
            mk(dst_slot, dst_slot, c, src_dev,
               s_z.at[0, 0], rpair[0].at[r, k]).wait_recv()

        S_R, S_L, S_Z = (s_r, t_r), (s_l, t_l), (s_z, t_z)
        R_L, R_R, R_Z = (r_l, u_l), (r_r, u_r), (r_z, u_z)

        for c in range(4):
            stage_comp(c)
            send(0, 4, c, partner, S_Z, R_Z, 0, c)
            send(0, 3, c, right, S_R, R_L, 0, c)
            send(0, 1, c, left, S_L, R_R, 0, c)
        compute(0, first=True)

        for c in range(4):
            recv(4, c, partner, R_Z, 0, c)
        send(4, 7, 0, right, S_R, R_L, 1, 0)
        send(4, 7, 1, right, S_R, R_L, 1, 1)
        send(4, 5, 2, left, S_L, R_R, 1, 0)
        send(4, 5, 3, left, S_L, R_R, 1, 1)
        for c in range(4):
            recv(3, c, left, R_L, 0, c)
        for c in range(4):
            recv(1, c, right, R_R, 0, c)
        send(3, 7, 2, partner, S_Z, R_Z, 1, 0)
        send(3, 7, 3, partner, S_Z, R_Z, 1, 1)
        send(1, 5, 0, partner, S_Z, R_Z, 1, 2)
        send(1, 5, 1, partner, S_Z, R_Z, 1, 3)
        send(3, 2, 0, right, S_R, R_L, 2, 0)
        send(3, 2, 1, right, S_R, R_L, 2, 1)
        send(1, 2, 2, left, S_L, R_R, 2, 0)
        send(1, 2, 3, left, S_L, R_R, 2, 1)
        compute(4)
        compute(3)
        compute(1)

        recv(7, 0, left, R_L, 1, 0)
        recv(7, 1, left, R_L, 1, 1)
        recv(7, 2, partner, R_Z, 1, 0)
        recv(7, 3, partner, R_Z, 1, 1)
        send(7, 6, 0, right, S_R, R_L, 2, 2)
        send(7, 6, 1, right, S_R, R_L, 2, 3)
        compute(7)
        recv(5, 2, right, R_R, 1, 0)
        recv(5, 3, right, R_R, 1, 1)
        recv(5, 0, partner, R_Z, 1, 2)
        recv(5, 1, partner, R_Z, 1, 3)
        send(5, 6, 2, left, S_L, R_R, 2, 2)
        send(5, 6, 3, left, S_L, R_R, 2, 3)
        compute(5)

        for k, (slot, c) in enumerate([(2, 0), (2, 1)]):
            recv(slot, c, left, R_L, 2, k)
        for k, (slot, c) in enumerate([(2, 2), (2, 3)]):
            recv(slot, c, right, R_R, 2, k)
        compute(2)
        for k, (slot, c) in enumerate([(6, 0), (6, 1)], start=2):
            recv(slot, c, left, R_L, 2, k)
        for k, (slot, c) in enumerate([(6, 2), (6, 3)], start=2):
            recv(slot, c, right, R_R, 2, k)
        compute(6)

        for d in sent:
            d.wait_send()

    out_shape = jax.ShapeDtypeStruct((B, SQ, D), f32)
    cosm = jnp.asarray(_COS, dtype=bf16)
    sinm = jnp.asarray(_SIN, dtype=bf16)
    rotm = jnp.asarray(_ROT, dtype=bf16)
    return pl.pallas_call(
        body,
        out_shape=out_shape,
        in_specs=[pl.BlockSpec(memory_space=pltpu.VMEM)] * 8,
        out_specs=pl.BlockSpec(memory_space=pltpu.VMEM),
        scratch_shapes=[
            pltpu.VMEM((BSQ, D), bf16),
            pltpu.VMEM((N_DEV, 3, D, CW), i8),
            pltpu.VMEM((N_DEV, CW, D), i8),
            pltpu.VMEM((N_DEV, 4, CW), f32),
            pltpu.SemaphoreType.DMA((3, 4)),
            pltpu.SemaphoreType.DMA((3, 4)),
            pltpu.SemaphoreType.DMA((3, 4)),
            pltpu.SemaphoreType.DMA((3, 4)),
            pltpu.SemaphoreType.DMA((2, 4)),
            pltpu.SemaphoreType.DMA((2, 4)),
            pltpu.SemaphoreType.DMA((3, 4)),
            pltpu.SemaphoreType.DMA((3, 4)),
            pltpu.SemaphoreType.DMA((3, 4)),
            pltpu.SemaphoreType.DMA((3, 4)),
            pltpu.SemaphoreType.DMA((2, 4)),
            pltpu.SemaphoreType.DMA((2, 4)),
        ],
        compiler_params=_CompilerParams(collective_id=0),
    )(x, Wq, Wk, Wv, Wo, cosm, sinm, rotm)
